# baseline (speedup 1.0000x reference)
import jax
import jax.numpy as jnp
from jax import lax
from jax.experimental import pallas as pl
from jax.experimental.pallas import tpu as pltpu
from jax.experimental.pallas import tpu_sc as plsc

_POS_T = 0.7
_NEG_T = 0.3
_NFG = 128
_NBG = 128
_N = 5000
_NPAD = 5120
_G = 64
_NSUB = 16
_PER = _NPAD // _NSUB
_VPER = _PER // 16
_L = 16


def _iota16():
    return lax.broadcasted_iota(jnp.int32, (_L,), 0)


def _splat_f(x):
    return jnp.full((_L,), x, jnp.float32)


def _splat_i(x):
    return jnp.full((_L,), x, jnp.int32)


def _lane0(vec):
    return jnp.sum(jnp.where(_iota16() == 0, vec, _splat_f(0.0)))


def _smooth_l1_v(v):
    av = jnp.abs(v)
    return jnp.where(av < 1.0, 0.5 * av * av, av - 0.5)


def _softplus_neg_abs(s):
    u = jnp.exp(-jnp.abs(s))
    z = u / (2.0 + u)
    z2 = z * z
    poly = 1.0 + z2 * (1.0 / 3.0 + z2 * (1.0 / 5.0 + z2 * (1.0 / 7.0 + z2 * (1.0 / 9.0))))
    return 2.0 * z * poly


def _loc_vec(rx_v, rowarg_v, gctr_v, v):
    sl = pl.ds(v * _L, _L)
    x1 = rx_v[pl.ds(0 * _PER + v * _L, _L)]
    y1 = rx_v[pl.ds(1 * _PER + v * _L, _L)]
    x2 = rx_v[pl.ds(2 * _PER + v * _L, _L)]
    y2 = rx_v[pl.ds(3 * _PER + v * _L, _L)]
    w = x2 - x1 + 1.0
    h = y2 - y1 + 1.0
    rcx = x1 + 0.5 * w
    rcy = y1 + 0.5 * h
    arg = rowarg_v[sl]
    tcx = plsc.load_gather(gctr_v, [arg])
    tcy = plsc.load_gather(gctr_v, [arg + _splat_i(_G)])
    tw = plsc.load_gather(gctr_v, [arg + _splat_i(2 * _G)])
    th = plsc.load_gather(gctr_v, [arg + _splat_i(3 * _G)])
    return (_smooth_l1_v(rcx - tcx) + _smooth_l1_v(rcy - tcy)
            + _smooth_l1_v(w - tw) + _smooth_l1_v(h - th))


def _bce_vecs(sc_v, v):
    s = sc_v[pl.ds(v * _L, _L)]
    sp = _softplus_neg_abs(s)
    relu = jnp.maximum(s, 0.0)
    return relu - s + sp, relu + sp


def _sc_body(cx1_hbm, cy1_hbm, cx2_hbm, cy2_hbm, scores_hbm, gts_hbm, out_hbm,
             rx_v, sc_v, gt_v, gctr_v, garea_v, area_v, iou_v, rowarg_v,
             colmax_v, pos_v, neg_v, allcm_v, allfin_v, stage_v, out_v,
             shared_cm, shared_cnt, shared_ls):
    sid = lax.axis_index("s")
    base = sid * _PER

    for c, ref in enumerate((cx1_hbm, cy1_hbm, cx2_hbm, cy2_hbm)):
        pltpu.sync_copy(ref.at[pl.ds(base, _PER)], rx_v.at[pl.ds(c * _PER, _PER)])
    pltpu.sync_copy(scores_hbm.at[pl.ds(base, _PER)], sc_v)
    pltpu.sync_copy(gts_hbm, gt_v)

    for jc in range(_G // _L):
        sl = pl.ds(jc * _L, _L)
        gx1 = gt_v[pl.ds(0 * _G + jc * _L, _L)]
        gy1 = gt_v[pl.ds(1 * _G + jc * _L, _L)]
        gx2 = gt_v[pl.ds(2 * _G + jc * _L, _L)]
        gy2 = gt_v[pl.ds(3 * _G + jc * _L, _L)]
        gw = gx2 - gx1 + 1.0
        gh = gy2 - gy1 + 1.0
        gctr_v[pl.ds(0 * _G + jc * _L, _L)] = gx1 + 0.5 * gw
        gctr_v[pl.ds(1 * _G + jc * _L, _L)] = gy1 + 0.5 * gh
        gctr_v[pl.ds(2 * _G + jc * _L, _L)] = gw
        gctr_v[pl.ds(3 * _G + jc * _L, _L)] = gh
        garea_v[sl] = gw * gh

    def area_body(v, carry):
        sl = pl.ds(v * _L, _L)
        x1 = rx_v[pl.ds(0 * _PER + v * _L, _L)]
        y1 = rx_v[pl.ds(1 * _PER + v * _L, _L)]
        x2 = rx_v[pl.ds(2 * _PER + v * _L, _L)]
        y2 = rx_v[pl.ds(3 * _PER + v * _L, _L)]
        area_v[sl] = (x2 - x1 + 1.0) * (y2 - y1 + 1.0)
        return carry

    lax.fori_loop(0, _VPER, area_body, 0)

    def jc_body(jc, carry):
        def jj_body(jj, cmvec):
            gc = []
            for k in range(4):
                j = jc * _L + jj * 4 + k
                jv = jnp.full((_L,), j, jnp.int32)
                gc.append((
                    plsc.load_gather(gt_v, [jv]),
                    plsc.load_gather(gt_v, [jv + _splat_i(_G)]),
                    plsc.load_gather(gt_v, [jv + _splat_i(2 * _G)]),
                    plsc.load_gather(gt_v, [jv + _splat_i(3 * _G)]),
                    plsc.load_gather(garea_v, [jv]),
                ))

            def v_body(vi, colaccs):
                colaccs = list(colaccs)
                for k2 in range(4):
                    v = vi * 4 + k2
                    sl = pl.ds(v * _L, _L)
                    x1 = rx_v[pl.ds(0 * _PER + v * _L, _L)]
                    y1 = rx_v[pl.ds(1 * _PER + v * _L, _L)]
                    x2 = rx_v[pl.ds(2 * _PER + v * _L, _L)]
                    y2 = rx_v[pl.ds(3 * _PER + v * _L, _L)]
                    area = area_v[sl]
                    for k in range(4):
                        gx1, gy1, gx2, gy2, garea = gc[k]
                        j = jc * _L + jj * 4 + k
                        iw = jnp.maximum(jnp.minimum(x2, gx2) - jnp.maximum(x1, gx1) + 1.0, 0.0)
                        ih = jnp.maximum(jnp.minimum(y2, gy2) - jnp.maximum(y1, gy1) + 1.0, 0.0)
                        inter = iw * ih
                        iou = inter / (area + garea - inter)
                        iou_v[pl.ds(j * _PER + v * _L, _L)] = iou
                        colaccs[k] = jnp.maximum(colaccs[k], iou)
                return tuple(colaccs)

            colaccs = lax.fori_loop(0, _VPER // 4, v_body,
                                    (_splat_f(0.0),) * 4)
            for k in range(4):
                cmj = jnp.max(colaccs[k])
                lane = jj * 4 + k
                cmvec = jnp.where(_iota16() == lane, jnp.full((_L,), cmj), cmvec)
            return cmvec

        cmvec = lax.fori_loop(0, _L // 4, jj_body, _splat_f(0.0))
        colmax_v[pl.ds(jc * _L, _L)] = cmvec
        return carry

    lax.fori_loop(0, _G // _L, jc_body, 0)

    pltpu.sync_copy(colmax_v, shared_cm.at[pl.ds(sid * _G, _G)])
    plsc.subcore_barrier()
    pltpu.sync_copy(shared_cm, allcm_v)
    for jc in range(_G // _L):

        def s_body(s, acc):
            return jnp.maximum(acc, allcm_v[pl.ds(s * _G + jc * _L, _L)])

        acc = lax.fori_loop(1, _NSUB, s_body, allcm_v[pl.ds(jc * _L, _L)])
        acc = jnp.where(acc == 0.0, _splat_f(1e-5), acc)
        colmax_v[pl.ds(jc * _L, _L)] = acc

    def lab_body(vi, carry):
        cntp_acc, cntn_acc = carry
        for k2 in range(2):
            v = vi * 2 + k2
            sl = pl.ds(v * _L, _L)

            def j_body(ji, st):
                rm, ra, kd = st
                for k in range(8):
                    j = ji * 8 + k
                    jv = jnp.full((_L,), j, jnp.int32)
                    iou = iou_v[pl.ds(j * _PER + v * _L, _L)]
                    cm = plsc.load_gather(colmax_v, [jv])
                    kd = jnp.maximum(kd, iou - cm)
                    upd = iou > rm
                    ra = jnp.where(upd, jv, ra)
                    rm = jnp.where(upd, iou, rm)
                return rm, ra, kd

            rm, ra, kd = lax.fori_loop(0, _G // 8, j_body,
                                       (_splat_f(-1.0), _splat_i(0),
                                        _splat_f(-1.0)))
            rowarg_v[sl] = ra
            ridx = base + v * _L + _iota16()
            valid = ridx < _N
            pos = ((kd == 0.0) | (rm > _POS_T)) & valid
            neg = (rm < _NEG_T) & (~pos) & valid
            posf = jnp.where(pos, _splat_f(1.0), _splat_f(0.0))
            negf = jnp.where(neg, _splat_f(1.0), _splat_f(0.0))
            pos_v[sl] = posf
            neg_v[sl] = negf
            cntp_acc = cntp_acc + posf
            cntn_acc = cntn_acc + negf
        return cntp_acc, cntn_acc

    cntp_acc, cntn_acc = lax.fori_loop(0, _VPER // 2, lab_body,
                                       (_splat_f(0.0), _splat_f(0.0)))
    cntp = jnp.sum(cntp_acc)
    cntn = jnp.sum(cntn_acc)
    it = _iota16()
    stage_v[...] = (jnp.where(it == 0, jnp.full((_L,), cntp), _splat_f(0.0))
                    + jnp.where(it == 1, jnp.full((_L,), cntn), _splat_f(0.0)))
    pltpu.sync_copy(stage_v, shared_cnt.at[pl.ds(sid * _L, _L)])
    plsc.subcore_barrier()

    pltpu.sync_copy(shared_cnt, allfin_v)
    cntp_vec = plsc.load_gather(allfin_v, [it * _L])
    cntn_vec = plsc.load_gather(allfin_v, [it * _L + _splat_i(1)])
    p_tot = jnp.sum(cntp_vec)
    n_tot = jnp.sum(cntn_vec)
    before = it < sid
    basep = jnp.sum(jnp.where(before, cntp_vec, _splat_f(0.0)))
    basen = jnp.sum(jnp.where(before, cntn_vec, _splat_f(0.0)))

    kp = jnp.minimum(p_tot, float(_NFG))
    kn = jnp.minimum(n_tot, float(_NBG))
    padp = float(_NFG) - kp
    padn = float(_NBG) - kn

    is0 = jnp.where(sid == 0, 1.0, 0.0)
    m0p = _lane0(pos_v[pl.ds(0, _L)])
    m0n = _lane0(neg_v[pl.ds(0, _L)])
    loc0 = _lane0(_loc_vec(rx_v, rowarg_v, gctr_v, 0))
    b1v0, b0v0 = _bce_vecs(sc_v, 0)
    b1_0 = _lane0(b1v0)
    b0_0 = _lane0(b0v0)
    corr_loc = is0 * padp * m0p * loc0
    corr_b1 = is0 * padp * m0p * b1_0
    corr_b0 = is0 * padn * m0n * b0_0

    def loss_body(v, carry):
        runp, runn, acc_loc, acc_b1, acc_b0 = carry
        sl = pl.ds(v * _L, _L)
        posf = pos_v[sl]
        negf = neg_v[sl]
        exclp = plsc.cumsum(posf) - posf
        excln = plsc.cumsum(negf) - negf
        rankp = exclp + jnp.full((_L,), basep + runp)
        rankn = excln + jnp.full((_L,), basen + runn)
        selp = jnp.where(rankp < float(_NFG), posf, _splat_f(0.0))
        seln = jnp.where(rankn < float(_NBG), negf, _splat_f(0.0))
        loc_i = _loc_vec(rx_v, rowarg_v, gctr_v, v)
        b1, b0 = _bce_vecs(sc_v, v)
        return (runp + jnp.sum(posf), runn + jnp.sum(negf),
                acc_loc + selp * loc_i, acc_b1 + selp * b1, acc_b0 + seln * b0)

    _, _, acc_loc, acc_b1, acc_b0 = lax.fori_loop(
        0, _VPER, loss_body,
        (0.0, 0.0, _splat_f(0.0), _splat_f(0.0), _splat_f(0.0)))
    loc_sum = jnp.sum(acc_loc) + corr_loc
    b1_sum = jnp.sum(acc_b1) + corr_b1
    b0_sum = jnp.sum(acc_b0) + corr_b0
    stage_v[...] = (jnp.where(it == 0, jnp.full((_L,), loc_sum), _splat_f(0.0))
                    + jnp.where(it == 1, jnp.full((_L,), b1_sum), _splat_f(0.0))
                    + jnp.where(it == 2, jnp.full((_L,), b0_sum), _splat_f(0.0)))
    pltpu.sync_copy(stage_v, shared_ls.at[pl.ds(sid * _L, _L)])
    plsc.subcore_barrier()

    pltpu.sync_copy(shared_ls, allfin_v)
    loc_num = jnp.sum(plsc.load_gather(allfin_v, [it * _L]))
    b1_num = jnp.sum(plsc.load_gather(allfin_v, [it * _L + _splat_i(1)]))
    b0_num = jnp.sum(plsc.load_gather(allfin_v, [it * _L + _splat_i(2)]))
    spw = jnp.full((_L,), kp + padp * m0p)
    snw = jnp.full((_L,), kn + padn * m0n)
    one = _splat_f(1.0)
    loc_loss = jnp.full((_L,), loc_num) / jnp.maximum(spw * 4.0, one)
    cls_p = jnp.full((_L,), b1_num) / jnp.maximum(spw, one)
    cls_n = jnp.full((_L,), b0_num) / jnp.maximum(snw, one)
    out_v[...] = loc_loss + cls_p + cls_n

    @pl.when(sid == 0)
    def _():
        pltpu.sync_copy(out_v, out_hbm)


def _make_sc_call():
    mesh = plsc.VectorSubcoreMesh(core_axis_name="c", subcore_axis_name="s",
                                  num_cores=1)
    return pl.kernel(
        _sc_body,
        mesh=mesh,
        compiler_params=pltpu.CompilerParams(needs_layout_passes=False),
        out_type=jax.ShapeDtypeStruct((_L,), jnp.float32),
        scratch_types=[
            pltpu.VMEM((4 * _PER,), jnp.float32),
            pltpu.VMEM((_PER,), jnp.float32),
            pltpu.VMEM((4 * _G,), jnp.float32),
            pltpu.VMEM((4 * _G,), jnp.float32),
            pltpu.VMEM((_G,), jnp.float32),
            pltpu.VMEM((_PER,), jnp.float32),
            pltpu.VMEM((_G * _PER,), jnp.float32),
            pltpu.VMEM((_PER,), jnp.int32),
            pltpu.VMEM((_G,), jnp.float32),
            pltpu.VMEM((_PER,), jnp.float32),
            pltpu.VMEM((_PER,), jnp.float32),
            pltpu.VMEM((_NSUB * _G,), jnp.float32),
            pltpu.VMEM((_NSUB * _L,), jnp.float32),
            pltpu.VMEM((_L,), jnp.float32),
            pltpu.VMEM((_L,), jnp.float32),
            pltpu.VMEM_SHARED((_NSUB * _G,), jnp.float32),
            pltpu.VMEM_SHARED((_NSUB * _L,), jnp.float32),
            pltpu.VMEM_SHARED((_NSUB * _L,), jnp.float32),
        ],
    )


_sc_call = _make_sc_call()


@jax.jit
def kernel(rois, fg_scores, gts):
    rois_pad = jnp.full((_NPAD, 4), -1e5, jnp.float32).at[:_N].set(rois)
    coords = rois_pad.T
    scores = jnp.pad(fg_scores[:, 0], (0, _NPAD - _N))
    gts_t = gts.T[:4].reshape(4 * _G)
    out = _sc_call(coords[0], coords[1], coords[2], coords[3], scores, gts_t)
    return out[0]


# speedup vs baseline: 1.0172x; 1.0172x over previous
import jax
import jax.numpy as jnp
from jax import lax
from jax.experimental import pallas as pl
from jax.experimental.pallas import tpu as pltpu
from jax.experimental.pallas import tpu_sc as plsc

_POS_T = 0.7
_NEG_T = 0.3
_NFG = 128
_NBG = 128
_N = 5000
_NPAD = 5120
_G = 64
_NSUB = 16
_PER = _NPAD // _NSUB
_VPER = _PER // 16
_L = 16


def _iota16():
    return lax.broadcasted_iota(jnp.int32, (_L,), 0)


def _splat_f(x):
    return jnp.full((_L,), x, jnp.float32)


def _splat_i(x):
    return jnp.full((_L,), x, jnp.int32)


def _lane0(vec):
    return jnp.sum(jnp.where(_iota16() == 0, vec, _splat_f(0.0)))


def _smooth_l1_v(v):
    av = jnp.abs(v)
    return jnp.where(av < 1.0, 0.5 * av * av, av - 0.5)


def _softplus_neg_abs(s):
    u = jnp.exp(-jnp.abs(s))
    z = u / (2.0 + u)
    z2 = z * z
    poly = 1.0 + z2 * (1.0 / 3.0 + z2 * (1.0 / 5.0 + z2 * (1.0 / 7.0 + z2 * (1.0 / 9.0))))
    return 2.0 * z * poly


def _loc_vec(rx_v, rowarg_v, gctr_v, v):
    sl = pl.ds(v * _L, _L)
    x1 = rx_v[pl.ds(0 * _PER + v * _L, _L)]
    y1 = rx_v[pl.ds(1 * _PER + v * _L, _L)]
    x2 = rx_v[pl.ds(2 * _PER + v * _L, _L)]
    y2 = rx_v[pl.ds(3 * _PER + v * _L, _L)]
    w = x2 - x1 + 1.0
    h = y2 - y1 + 1.0
    rcx = x1 + 0.5 * w
    rcy = y1 + 0.5 * h
    arg = rowarg_v[sl]
    tcx = plsc.load_gather(gctr_v, [arg])
    tcy = plsc.load_gather(gctr_v, [arg + _splat_i(_G)])
    tw = plsc.load_gather(gctr_v, [arg + _splat_i(2 * _G)])
    th = plsc.load_gather(gctr_v, [arg + _splat_i(3 * _G)])
    return (_smooth_l1_v(rcx - tcx) + _smooth_l1_v(rcy - tcy)
            + _smooth_l1_v(w - tw) + _smooth_l1_v(h - th))


def _bce_vecs(sc_v, v):
    s = sc_v[pl.ds(v * _L, _L)]
    sp = _softplus_neg_abs(s)
    relu = jnp.maximum(s, 0.0)
    return relu - s + sp, relu + sp


def _sc_body(cx1_hbm, cy1_hbm, cx2_hbm, cy2_hbm, scores_hbm, gts_hbm, out_hbm,
             rx_v, sc_v, gt_v, gctr_v, garea_v, area_v, iou_v, rowarg_v,
             colmax_v, pos_v, neg_v, allcm_v, allfin_v, stage_v, out_v,
             shared_cm, shared_cnt, shared_ls):
    sid = lax.axis_index("s")
    base = sid * _PER

    for c, ref in enumerate((cx1_hbm, cy1_hbm, cx2_hbm, cy2_hbm)):
        pltpu.sync_copy(ref.at[pl.ds(base, _PER)], rx_v.at[pl.ds(c * _PER, _PER)])
    pltpu.sync_copy(scores_hbm.at[pl.ds(base, _PER)], sc_v)
    pltpu.sync_copy(gts_hbm, gt_v)

    for jc in range(_G // _L):
        sl = pl.ds(jc * _L, _L)
        gx1 = gt_v[pl.ds(0 * _G + jc * _L, _L)]
        gy1 = gt_v[pl.ds(1 * _G + jc * _L, _L)]
        gx2 = gt_v[pl.ds(2 * _G + jc * _L, _L)]
        gy2 = gt_v[pl.ds(3 * _G + jc * _L, _L)]
        gw = gx2 - gx1 + 1.0
        gh = gy2 - gy1 + 1.0
        gctr_v[pl.ds(0 * _G + jc * _L, _L)] = gx1 + 0.5 * gw
        gctr_v[pl.ds(1 * _G + jc * _L, _L)] = gy1 + 0.5 * gh
        gctr_v[pl.ds(2 * _G + jc * _L, _L)] = gw
        gctr_v[pl.ds(3 * _G + jc * _L, _L)] = gh
        garea_v[sl] = gw * gh

    def area_body(v, carry):
        sl = pl.ds(v * _L, _L)
        x1 = rx_v[pl.ds(0 * _PER + v * _L, _L)]
        y1 = rx_v[pl.ds(1 * _PER + v * _L, _L)]
        x2 = rx_v[pl.ds(2 * _PER + v * _L, _L)]
        y2 = rx_v[pl.ds(3 * _PER + v * _L, _L)]
        area_v[sl] = (x2 - x1 + 1.0) * (y2 - y1 + 1.0)
        return carry

    lax.fori_loop(0, _VPER, area_body, 0)

    def jc_body(jc, carry):
        def jj_body(jj, cmvec):
            gc = []
            for k in range(4):
                j = jc * _L + jj * 4 + k
                jv = jnp.full((_L,), j, jnp.int32)
                gc.append((
                    plsc.load_gather(gt_v, [jv]),
                    plsc.load_gather(gt_v, [jv + _splat_i(_G)]),
                    plsc.load_gather(gt_v, [jv + _splat_i(2 * _G)]),
                    plsc.load_gather(gt_v, [jv + _splat_i(3 * _G)]),
                    plsc.load_gather(garea_v, [jv]),
                ))

            def v_body(vi, colaccs):
                colaccs = list(colaccs)
                for k2 in range(2):
                    v = vi * 2 + k2
                    sl = pl.ds(v * _L, _L)
                    x1 = rx_v[pl.ds(0 * _PER + v * _L, _L)]
                    y1 = rx_v[pl.ds(1 * _PER + v * _L, _L)]
                    x2 = rx_v[pl.ds(2 * _PER + v * _L, _L)]
                    y2 = rx_v[pl.ds(3 * _PER + v * _L, _L)]
                    area = area_v[sl]
                    for k in range(4):
                        gx1, gy1, gx2, gy2, garea = gc[k]
                        j = jc * _L + jj * 4 + k
                        iw = jnp.maximum(jnp.minimum(x2, gx2) - jnp.maximum(x1, gx1) + 1.0, 0.0)
                        ih = jnp.maximum(jnp.minimum(y2, gy2) - jnp.maximum(y1, gy1) + 1.0, 0.0)
                        inter = iw * ih
                        iou = inter / (area + garea - inter)
                        iou_v[pl.ds(j * _PER + v * _L, _L)] = iou
                        colaccs[k] = jnp.maximum(colaccs[k], iou)
                return tuple(colaccs)

            colaccs = lax.fori_loop(0, _VPER // 2, v_body,
                                    (_splat_f(0.0),) * 4)
            for k in range(4):
                cmj = jnp.max(colaccs[k])
                lane = jj * 4 + k
                cmvec = jnp.where(_iota16() == lane, jnp.full((_L,), cmj), cmvec)
            return cmvec

        cmvec = lax.fori_loop(0, _L // 4, jj_body, _splat_f(0.0))
        colmax_v[pl.ds(jc * _L, _L)] = cmvec
        return carry

    lax.fori_loop(0, _G // _L, jc_body, 0)

    pltpu.sync_copy(colmax_v, shared_cm.at[pl.ds(sid * _G, _G)])
    plsc.subcore_barrier()
    pltpu.sync_copy(shared_cm, allcm_v)
    for jc in range(_G // _L):

        def s_body(s, acc):
            return jnp.maximum(acc, allcm_v[pl.ds(s * _G + jc * _L, _L)])

        acc = lax.fori_loop(1, _NSUB, s_body, allcm_v[pl.ds(jc * _L, _L)])
        acc = jnp.where(acc == 0.0, _splat_f(1e-5), acc)
        colmax_v[pl.ds(jc * _L, _L)] = acc

    def lab_body(vi, carry):
        cntp_acc, cntn_acc = carry
        for k2 in range(2):
            v = vi * 2 + k2
            sl = pl.ds(v * _L, _L)

            def j_body(ji, st):
                rm, ra, kd = st
                for k in range(8):
                    j = ji * 8 + k
                    jv = jnp.full((_L,), j, jnp.int32)
                    iou = iou_v[pl.ds(j * _PER + v * _L, _L)]
                    cm = plsc.load_gather(colmax_v, [jv])
                    kd = jnp.maximum(kd, iou - cm)
                    upd = iou > rm
                    ra = jnp.where(upd, jv, ra)
                    rm = jnp.where(upd, iou, rm)
                return rm, ra, kd

            rm, ra, kd = lax.fori_loop(0, _G // 8, j_body,
                                       (_splat_f(-1.0), _splat_i(0),
                                        _splat_f(-1.0)))
            rowarg_v[sl] = ra
            ridx = base + v * _L + _iota16()
            valid = ridx < _N
            pos = ((kd == 0.0) | (rm > _POS_T)) & valid
            neg = (rm < _NEG_T) & (~pos) & valid
            posf = jnp.where(pos, _splat_f(1.0), _splat_f(0.0))
            negf = jnp.where(neg, _splat_f(1.0), _splat_f(0.0))
            pos_v[sl] = posf
            neg_v[sl] = negf
            cntp_acc = cntp_acc + posf
            cntn_acc = cntn_acc + negf
        return cntp_acc, cntn_acc

    cntp_acc, cntn_acc = lax.fori_loop(0, _VPER // 2, lab_body,
                                       (_splat_f(0.0), _splat_f(0.0)))
    cntp = jnp.sum(cntp_acc)
    cntn = jnp.sum(cntn_acc)
    it = _iota16()
    stage_v[...] = (jnp.where(it == 0, jnp.full((_L,), cntp), _splat_f(0.0))
                    + jnp.where(it == 1, jnp.full((_L,), cntn), _splat_f(0.0)))
    pltpu.sync_copy(stage_v, shared_cnt.at[pl.ds(sid * _L, _L)])
    plsc.subcore_barrier()

    pltpu.sync_copy(shared_cnt, allfin_v)
    cntp_vec = plsc.load_gather(allfin_v, [it * _L])
    cntn_vec = plsc.load_gather(allfin_v, [it * _L + _splat_i(1)])
    p_tot = jnp.sum(cntp_vec)
    n_tot = jnp.sum(cntn_vec)
    before = it < sid
    basep = jnp.sum(jnp.where(before, cntp_vec, _splat_f(0.0)))
    basen = jnp.sum(jnp.where(before, cntn_vec, _splat_f(0.0)))

    kp = jnp.minimum(p_tot, float(_NFG))
    kn = jnp.minimum(n_tot, float(_NBG))
    padp = float(_NFG) - kp
    padn = float(_NBG) - kn

    is0 = jnp.where(sid == 0, 1.0, 0.0)
    m0p = _lane0(pos_v[pl.ds(0, _L)])
    m0n = _lane0(neg_v[pl.ds(0, _L)])
    loc0 = _lane0(_loc_vec(rx_v, rowarg_v, gctr_v, 0))
    b1v0, b0v0 = _bce_vecs(sc_v, 0)
    b1_0 = _lane0(b1v0)
    b0_0 = _lane0(b0v0)
    corr_loc = is0 * padp * m0p * loc0
    corr_b1 = is0 * padp * m0p * b1_0
    corr_b0 = is0 * padn * m0n * b0_0

    def loss_body(v, carry):
        runp, runn, acc_loc, acc_b1, acc_b0 = carry
        sl = pl.ds(v * _L, _L)
        posf = pos_v[sl]
        negf = neg_v[sl]
        exclp = plsc.cumsum(posf) - posf
        excln = plsc.cumsum(negf) - negf
        rankp = exclp + jnp.full((_L,), basep + runp)
        rankn = excln + jnp.full((_L,), basen + runn)
        selp = jnp.where(rankp < float(_NFG), posf, _splat_f(0.0))
        seln = jnp.where(rankn < float(_NBG), negf, _splat_f(0.0))
        loc_i = _loc_vec(rx_v, rowarg_v, gctr_v, v)
        b1, b0 = _bce_vecs(sc_v, v)
        return (runp + jnp.sum(posf), runn + jnp.sum(negf),
                acc_loc + selp * loc_i, acc_b1 + selp * b1, acc_b0 + seln * b0)

    _, _, acc_loc, acc_b1, acc_b0 = lax.fori_loop(
        0, _VPER, loss_body,
        (0.0, 0.0, _splat_f(0.0), _splat_f(0.0), _splat_f(0.0)))
    loc_sum = jnp.sum(acc_loc) + corr_loc
    b1_sum = jnp.sum(acc_b1) + corr_b1
    b0_sum = jnp.sum(acc_b0) + corr_b0
    stage_v[...] = (jnp.where(it == 0, jnp.full((_L,), loc_sum), _splat_f(0.0))
                    + jnp.where(it == 1, jnp.full((_L,), b1_sum), _splat_f(0.0))
                    + jnp.where(it == 2, jnp.full((_L,), b0_sum), _splat_f(0.0)))
    pltpu.sync_copy(stage_v, shared_ls.at[pl.ds(sid * _L, _L)])
    plsc.subcore_barrier()

    pltpu.sync_copy(shared_ls, allfin_v)
    loc_num = jnp.sum(plsc.load_gather(allfin_v, [it * _L]))
    b1_num = jnp.sum(plsc.load_gather(allfin_v, [it * _L + _splat_i(1)]))
    b0_num = jnp.sum(plsc.load_gather(allfin_v, [it * _L + _splat_i(2)]))
    spw = jnp.full((_L,), kp + padp * m0p)
    snw = jnp.full((_L,), kn + padn * m0n)
    one = _splat_f(1.0)
    loc_loss = jnp.full((_L,), loc_num) / jnp.maximum(spw * 4.0, one)
    cls_p = jnp.full((_L,), b1_num) / jnp.maximum(spw, one)
    cls_n = jnp.full((_L,), b0_num) / jnp.maximum(snw, one)
    out_v[...] = loc_loss + cls_p + cls_n

    @pl.when(sid == 0)
    def _():
        pltpu.sync_copy(out_v, out_hbm)


def _make_sc_call():
    mesh = plsc.VectorSubcoreMesh(core_axis_name="c", subcore_axis_name="s",
                                  num_cores=1)
    return pl.kernel(
        _sc_body,
        mesh=mesh,
        compiler_params=pltpu.CompilerParams(needs_layout_passes=False),
        out_type=jax.ShapeDtypeStruct((_L,), jnp.float32),
        scratch_types=[
            pltpu.VMEM((4 * _PER,), jnp.float32),
            pltpu.VMEM((_PER,), jnp.float32),
            pltpu.VMEM((4 * _G,), jnp.float32),
            pltpu.VMEM((4 * _G,), jnp.float32),
            pltpu.VMEM((_G,), jnp.float32),
            pltpu.VMEM((_PER,), jnp.float32),
            pltpu.VMEM((_G * _PER,), jnp.float32),
            pltpu.VMEM((_PER,), jnp.int32),
            pltpu.VMEM((_G,), jnp.float32),
            pltpu.VMEM((_PER,), jnp.float32),
            pltpu.VMEM((_PER,), jnp.float32),
            pltpu.VMEM((_NSUB * _G,), jnp.float32),
            pltpu.VMEM((_NSUB * _L,), jnp.float32),
            pltpu.VMEM((_L,), jnp.float32),
            pltpu.VMEM((_L,), jnp.float32),
            pltpu.VMEM_SHARED((_NSUB * _G,), jnp.float32),
            pltpu.VMEM_SHARED((_NSUB * _L,), jnp.float32),
            pltpu.VMEM_SHARED((_NSUB * _L,), jnp.float32),
        ],
    )


_sc_call = _make_sc_call()


@jax.jit
def kernel(rois, fg_scores, gts):
    rois_pad = jnp.full((_NPAD, 4), -1e5, jnp.float32).at[:_N].set(rois)
    coords = rois_pad.T
    scores = jnp.pad(fg_scores[:, 0], (0, _NPAD - _N))
    gts_t = gts.T[:4].reshape(4 * _G)
    out = _sc_call(coords[0], coords[1], coords[2], coords[3], scores, gts_t)
    return out[0]


# speedup vs baseline: 1.0815x; 1.0632x over previous
import jax
import jax.numpy as jnp
from jax import lax
from jax.experimental import pallas as pl
from jax.experimental.pallas import tpu as pltpu
from jax.experimental.pallas import tpu_sc as plsc

_POS_T = 0.7
_NEG_T = 0.3
_NFG = 128
_NBG = 128
_N = 5000
_NPAD = 5120
_G = 64
_NSUB = 16
_PER = _NPAD // _NSUB
_VPER = _PER // 16
_L = 16


def _iota16():
    return lax.broadcasted_iota(jnp.int32, (_L,), 0)


def _splat_f(x):
    return jnp.full((_L,), x, jnp.float32)


def _splat_i(x):
    return jnp.full((_L,), x, jnp.int32)


def _lane0(vec):
    return jnp.sum(jnp.where(_iota16() == 0, vec, _splat_f(0.0)))


def _smooth_l1_v(v):
    av = jnp.abs(v)
    return jnp.where(av < 1.0, 0.5 * av * av, av - 0.5)


def _softplus_neg_abs(s):
    u = jnp.exp(-jnp.abs(s))
    z = u / (2.0 + u)
    z2 = z * z
    poly = 1.0 + z2 * (1.0 / 3.0 + z2 * (1.0 / 5.0 + z2 * (1.0 / 7.0 + z2 * (1.0 / 9.0))))
    return 2.0 * z * poly


def _loc_vec(rx_v, rowarg_v, gctr_v, v):
    sl = pl.ds(v * _L, _L)
    x1 = rx_v[pl.ds(0 * _PER + v * _L, _L)]
    y1 = rx_v[pl.ds(1 * _PER + v * _L, _L)]
    x2 = rx_v[pl.ds(2 * _PER + v * _L, _L)]
    y2 = rx_v[pl.ds(3 * _PER + v * _L, _L)]
    w = x2 - x1 + 1.0
    h = y2 - y1 + 1.0
    rcx = x1 + 0.5 * w
    rcy = y1 + 0.5 * h
    arg = rowarg_v[sl]
    tcx = plsc.load_gather(gctr_v, [arg])
    tcy = plsc.load_gather(gctr_v, [arg + _splat_i(_G)])
    tw = plsc.load_gather(gctr_v, [arg + _splat_i(2 * _G)])
    th = plsc.load_gather(gctr_v, [arg + _splat_i(3 * _G)])
    return (_smooth_l1_v(rcx - tcx) + _smooth_l1_v(rcy - tcy)
            + _smooth_l1_v(w - tw) + _smooth_l1_v(h - th))


def _bce_vecs(sc_v, v):
    s = sc_v[pl.ds(v * _L, _L)]
    sp = _softplus_neg_abs(s)
    relu = jnp.maximum(s, 0.0)
    return relu - s + sp, relu + sp


def _sc_body(cx1_hbm, cy1_hbm, cx2_hbm, cy2_hbm, scores_hbm, gts_hbm, out_hbm,
             rx_v, sc_v, gt_v, gctr_v, garea_v, area_v, iou_v, rowarg_v,
             colmax_v, pos_v, neg_v, allcm_v, allfin_v, stage_v, out_v,
             shared_cm, shared_cnt, shared_ls):
    sid = lax.axis_index("s")
    base = sid * _PER

    for c, ref in enumerate((cx1_hbm, cy1_hbm, cx2_hbm, cy2_hbm)):
        pltpu.sync_copy(ref.at[pl.ds(base, _PER)], rx_v.at[pl.ds(c * _PER, _PER)])
    pltpu.sync_copy(scores_hbm.at[pl.ds(base, _PER)], sc_v)
    pltpu.sync_copy(gts_hbm, gt_v)

    for jc in range(_G // _L):
        sl = pl.ds(jc * _L, _L)
        gx1 = gt_v[pl.ds(0 * _G + jc * _L, _L)]
        gy1 = gt_v[pl.ds(1 * _G + jc * _L, _L)]
        gx2 = gt_v[pl.ds(2 * _G + jc * _L, _L)]
        gy2 = gt_v[pl.ds(3 * _G + jc * _L, _L)]
        gw = gx2 - gx1 + 1.0
        gh = gy2 - gy1 + 1.0
        gctr_v[pl.ds(0 * _G + jc * _L, _L)] = gx1 + 0.5 * gw
        gctr_v[pl.ds(1 * _G + jc * _L, _L)] = gy1 + 0.5 * gh
        gctr_v[pl.ds(2 * _G + jc * _L, _L)] = gw
        gctr_v[pl.ds(3 * _G + jc * _L, _L)] = gh
        garea_v[sl] = gw * gh

    def area_body(v, carry):
        sl = pl.ds(v * _L, _L)
        x1 = rx_v[pl.ds(0 * _PER + v * _L, _L)]
        y1 = rx_v[pl.ds(1 * _PER + v * _L, _L)]
        x2 = rx_v[pl.ds(2 * _PER + v * _L, _L)]
        y2 = rx_v[pl.ds(3 * _PER + v * _L, _L)]
        area_v[sl] = (x2 - x1 + 1.0) * (y2 - y1 + 1.0)
        return carry

    lax.fori_loop(0, _VPER, area_body, 0)

    def jc_body(jc, carry):
        def jj_body(jj, cmvec):
            gc = []
            for k in range(4):
                j = jc * _L + jj * 4 + k
                jv = jnp.full((_L,), j, jnp.int32)
                gc.append((
                    plsc.load_gather(gt_v, [jv]),
                    plsc.load_gather(gt_v, [jv + _splat_i(_G)]),
                    plsc.load_gather(gt_v, [jv + _splat_i(2 * _G)]),
                    plsc.load_gather(gt_v, [jv + _splat_i(3 * _G)]),
                    plsc.load_gather(garea_v, [jv]),
                ))

            @plsc.parallel_loop(0, _VPER // 2, carry=(_splat_f(0.0),) * 4,
                                unroll=2)
            def colaccs(vi, colaccs):
                colaccs = list(colaccs)
                for k2 in range(2):
                    v = vi * 2 + k2
                    sl = pl.ds(v * _L, _L)
                    x1 = rx_v[pl.ds(0 * _PER + v * _L, _L)]
                    y1 = rx_v[pl.ds(1 * _PER + v * _L, _L)]
                    x2 = rx_v[pl.ds(2 * _PER + v * _L, _L)]
                    y2 = rx_v[pl.ds(3 * _PER + v * _L, _L)]
                    area = area_v[sl]
                    for k in range(4):
                        gx1, gy1, gx2, gy2, garea = gc[k]
                        j = jc * _L + jj * 4 + k
                        iw = jnp.maximum(jnp.minimum(x2, gx2) - jnp.maximum(x1, gx1) + 1.0, 0.0)
                        ih = jnp.maximum(jnp.minimum(y2, gy2) - jnp.maximum(y1, gy1) + 1.0, 0.0)
                        inter = iw * ih
                        iou = inter / (area + garea - inter)
                        iou_v[pl.ds(j * _PER + v * _L, _L)] = iou
                        colaccs[k] = jnp.maximum(colaccs[k], iou)
                return tuple(colaccs)

            for k in range(4):
                cmj = jnp.max(colaccs[k])
                lane = jj * 4 + k
                cmvec = jnp.where(_iota16() == lane, jnp.full((_L,), cmj), cmvec)
            return cmvec

        cmvec = lax.fori_loop(0, _L // 4, jj_body, _splat_f(0.0))
        colmax_v[pl.ds(jc * _L, _L)] = cmvec
        return carry

    lax.fori_loop(0, _G // _L, jc_body, 0)

    pltpu.sync_copy(colmax_v, shared_cm.at[pl.ds(sid * _G, _G)])
    plsc.subcore_barrier()
    pltpu.sync_copy(shared_cm, allcm_v)
    for jc in range(_G // _L):

        def s_body(s, acc):
            return jnp.maximum(acc, allcm_v[pl.ds(s * _G + jc * _L, _L)])

        acc = lax.fori_loop(1, _NSUB, s_body, allcm_v[pl.ds(jc * _L, _L)])
        acc = jnp.where(acc == 0.0, _splat_f(1e-5), acc)
        colmax_v[pl.ds(jc * _L, _L)] = acc

    def lab_body(vi, carry):
        cntp_acc, cntn_acc = carry
        for k2 in range(2):
            v = vi * 2 + k2
            sl = pl.ds(v * _L, _L)

            @plsc.parallel_loop(0, _G // 8,
                                carry=(_splat_f(-1.0), _splat_i(0),
                                       _splat_f(-1.0)),
                                unroll=2)
            def st(ji, st):
                rm, ra, kd = st
                for k in range(8):
                    j = ji * 8 + k
                    jv = jnp.full((_L,), j, jnp.int32)
                    iou = iou_v[pl.ds(j * _PER + v * _L, _L)]
                    cm = plsc.load_gather(colmax_v, [jv])
                    kd = jnp.maximum(kd, iou - cm)
                    upd = iou > rm
                    ra = jnp.where(upd, jv, ra)
                    rm = jnp.where(upd, iou, rm)
                return rm, ra, kd

            rm, ra, kd = st
            rowarg_v[sl] = ra
            ridx = base + v * _L + _iota16()
            valid = ridx < _N
            pos = ((kd == 0.0) | (rm > _POS_T)) & valid
            neg = (rm < _NEG_T) & (~pos) & valid
            posf = jnp.where(pos, _splat_f(1.0), _splat_f(0.0))
            negf = jnp.where(neg, _splat_f(1.0), _splat_f(0.0))
            pos_v[sl] = posf
            neg_v[sl] = negf
            cntp_acc = cntp_acc + posf
            cntn_acc = cntn_acc + negf
        return cntp_acc, cntn_acc

    cntp_acc, cntn_acc = lax.fori_loop(0, _VPER // 2, lab_body,
                                       (_splat_f(0.0), _splat_f(0.0)))
    cntp = jnp.sum(cntp_acc)
    cntn = jnp.sum(cntn_acc)
    it = _iota16()
    stage_v[...] = (jnp.where(it == 0, jnp.full((_L,), cntp), _splat_f(0.0))
                    + jnp.where(it == 1, jnp.full((_L,), cntn), _splat_f(0.0)))
    pltpu.sync_copy(stage_v, shared_cnt.at[pl.ds(sid * _L, _L)])
    plsc.subcore_barrier()

    pltpu.sync_copy(shared_cnt, allfin_v)
    cntp_vec = plsc.load_gather(allfin_v, [it * _L])
    cntn_vec = plsc.load_gather(allfin_v, [it * _L + _splat_i(1)])
    p_tot = jnp.sum(cntp_vec)
    n_tot = jnp.sum(cntn_vec)
    before = it < sid
    basep = jnp.sum(jnp.where(before, cntp_vec, _splat_f(0.0)))
    basen = jnp.sum(jnp.where(before, cntn_vec, _splat_f(0.0)))

    kp = jnp.minimum(p_tot, float(_NFG))
    kn = jnp.minimum(n_tot, float(_NBG))
    padp = float(_NFG) - kp
    padn = float(_NBG) - kn

    is0 = jnp.where(sid == 0, 1.0, 0.0)
    m0p = _lane0(pos_v[pl.ds(0, _L)])
    m0n = _lane0(neg_v[pl.ds(0, _L)])
    loc0 = _lane0(_loc_vec(rx_v, rowarg_v, gctr_v, 0))
    b1v0, b0v0 = _bce_vecs(sc_v, 0)
    b1_0 = _lane0(b1v0)
    b0_0 = _lane0(b0v0)
    corr_loc = is0 * padp * m0p * loc0
    corr_b1 = is0 * padp * m0p * b1_0
    corr_b0 = is0 * padn * m0n * b0_0

    def loss_body(v, carry):
        runp, runn, acc_loc, acc_b1, acc_b0 = carry
        sl = pl.ds(v * _L, _L)
        posf = pos_v[sl]
        negf = neg_v[sl]
        exclp = plsc.cumsum(posf) - posf
        excln = plsc.cumsum(negf) - negf
        rankp = exclp + jnp.full((_L,), basep + runp)
        rankn = excln + jnp.full((_L,), basen + runn)
        selp = jnp.where(rankp < float(_NFG), posf, _splat_f(0.0))
        seln = jnp.where(rankn < float(_NBG), negf, _splat_f(0.0))
        loc_i = _loc_vec(rx_v, rowarg_v, gctr_v, v)
        b1, b0 = _bce_vecs(sc_v, v)
        return (runp + jnp.sum(posf), runn + jnp.sum(negf),
                acc_loc + selp * loc_i, acc_b1 + selp * b1, acc_b0 + seln * b0)

    _, _, acc_loc, acc_b1, acc_b0 = lax.fori_loop(
        0, _VPER, loss_body,
        (0.0, 0.0, _splat_f(0.0), _splat_f(0.0), _splat_f(0.0)))
    loc_sum = jnp.sum(acc_loc) + corr_loc
    b1_sum = jnp.sum(acc_b1) + corr_b1
    b0_sum = jnp.sum(acc_b0) + corr_b0
    stage_v[...] = (jnp.where(it == 0, jnp.full((_L,), loc_sum), _splat_f(0.0))
                    + jnp.where(it == 1, jnp.full((_L,), b1_sum), _splat_f(0.0))
                    + jnp.where(it == 2, jnp.full((_L,), b0_sum), _splat_f(0.0)))
    pltpu.sync_copy(stage_v, shared_ls.at[pl.ds(sid * _L, _L)])
    plsc.subcore_barrier()

    pltpu.sync_copy(shared_ls, allfin_v)
    loc_num = jnp.sum(plsc.load_gather(allfin_v, [it * _L]))
    b1_num = jnp.sum(plsc.load_gather(allfin_v, [it * _L + _splat_i(1)]))
    b0_num = jnp.sum(plsc.load_gather(allfin_v, [it * _L + _splat_i(2)]))
    spw = jnp.full((_L,), kp + padp * m0p)
    snw = jnp.full((_L,), kn + padn * m0n)
    one = _splat_f(1.0)
    loc_loss = jnp.full((_L,), loc_num) / jnp.maximum(spw * 4.0, one)
    cls_p = jnp.full((_L,), b1_num) / jnp.maximum(spw, one)
    cls_n = jnp.full((_L,), b0_num) / jnp.maximum(snw, one)
    out_v[...] = loc_loss + cls_p + cls_n

    @pl.when(sid == 0)
    def _():
        pltpu.sync_copy(out_v, out_hbm)


def _make_sc_call():
    mesh = plsc.VectorSubcoreMesh(core_axis_name="c", subcore_axis_name="s",
                                  num_cores=1)
    return pl.kernel(
        _sc_body,
        mesh=mesh,
        compiler_params=pltpu.CompilerParams(needs_layout_passes=False),
        out_type=jax.ShapeDtypeStruct((_L,), jnp.float32),
        scratch_types=[
            pltpu.VMEM((4 * _PER,), jnp.float32),
            pltpu.VMEM((_PER,), jnp.float32),
            pltpu.VMEM((4 * _G,), jnp.float32),
            pltpu.VMEM((4 * _G,), jnp.float32),
            pltpu.VMEM((_G,), jnp.float32),
            pltpu.VMEM((_PER,), jnp.float32),
            pltpu.VMEM((_G * _PER,), jnp.float32),
            pltpu.VMEM((_PER,), jnp.int32),
            pltpu.VMEM((_G,), jnp.float32),
            pltpu.VMEM((_PER,), jnp.float32),
            pltpu.VMEM((_PER,), jnp.float32),
            pltpu.VMEM((_NSUB * _G,), jnp.float32),
            pltpu.VMEM((_NSUB * _L,), jnp.float32),
            pltpu.VMEM((_L,), jnp.float32),
            pltpu.VMEM((_L,), jnp.float32),
            pltpu.VMEM_SHARED((_NSUB * _G,), jnp.float32),
            pltpu.VMEM_SHARED((_NSUB * _L,), jnp.float32),
            pltpu.VMEM_SHARED((_NSUB * _L,), jnp.float32),
        ],
    )


_sc_call = _make_sc_call()


@jax.jit
def kernel(rois, fg_scores, gts):
    rois_pad = jnp.full((_NPAD, 4), -1e5, jnp.float32).at[:_N].set(rois)
    coords = rois_pad.T
    scores = jnp.pad(fg_scores[:, 0], (0, _NPAD - _N))
    gts_t = gts.T[:4].reshape(4 * _G)
    out = _sc_call(coords[0], coords[1], coords[2], coords[3], scores, gts_t)
    return out[0]
